# Initial kernel scaffold; baseline (speedup 1.0000x reference)
import jax
import jax.numpy as jnp
from jax.experimental import pallas as pl


def kernel(x, edge_index, batch, W1, b1, W2, b2, gamma1, beta1, p1, W3, b3, W4, b4, gamma2, beta2, p2, Wl, bl):
    raise NotImplementedError("write your pallas kernel here")


# speedup vs baseline: 4.2663x; 4.2663x over previous
import functools

import jax
import jax.numpy as jnp
from jax import lax
from jax.experimental import pallas as pl
from jax.experimental.pallas import tpu as pltpu
from jax.experimental.pallas import tpu_sc as plsc

N = 10000
E = 320000
DIN = 128
H = 256
K1 = 5000
K2 = 2500
NP = 10240
K1P = 5120
RB = 512
RB2 = 256
NB1 = NP // RB
NB2 = K1P // RB2
NC = 2
NS = 16
QN1 = NP // 4
A1R = QN1 + 128
QN2 = 1536
A2R = QN2 + 128
EPW = E // NS
EPW_PAD = 20096
CH = 128
EC1 = 20224
EC2 = 20864
NPW = NP // NS
KEY_BIAS = 0x3F800002


SUP = 512
NSUP = EPW // SUP
TAIL = EPW - NSUP * SUP
GB = 64
RS1 = 576
RS2 = 1088
RSS = 1408
CROW1 = QN1 // NS
CROW2 = 2 * QN2 // NS
NCNT = 48


def _store_cnts(cbuf, cnts, lane):
    for t in range(NCNT // 16):
        vec = jnp.zeros((16,), jnp.int32)
        for i in range(16):
            idx = t * 16 + i
            if idx < len(cnts):
                vec = jnp.where(lane == i, cnts[idx], vec)
        cbuf[pl.ds(t * 16, 16)] = vec


def _zero_acc(acc, nrows):
    zv = jnp.zeros((16,), jnp.float32)

    def zbody(r, c):
        for t in range(DIN // 16):
            acc[r, pl.ds(t * 16, 16)] = zv
        return c

    lax.fori_loop(0, nrows, zbody, 0)


def _compact_chunk(sbuf, dbuf, es, er, nit, keepfn, cnt):
    def cbody(i, cnt):
        sv = sbuf[pl.ds(i * 16, 16)]
        dv = dbuf[pl.ds(i * 16, 16)]
        return keepfn(sv, dv, es, er, cnt)

    return lax.fori_loop(0, nit, cbody, cnt)


def _flush(es, er, eshbm, erhbm, base, cnt, lane, cap):
    for t in range(GB // 16):
        es[pl.ds(cnt + t * 16, 16)] = lane + t * 16
        er[pl.ds(cnt + t * 16, 16)] = jnp.full((16,), 1 << 20, jnp.int32)
    pltpu.sync_copy(es.at[pl.ds(0, cap)], eshbm.at[pl.ds(base, cap)])
    pltpu.sync_copy(er.at[pl.ds(0, cap)], erhbm.at[pl.ds(base, cap)])


def _fire(table, acc, pes, plr, rows, pcnt, shift):
    pltpu.sync_copy(table.at[pes.at[pl.ds(0, GB)]], rows)

    def abody(k, c):
        l = plr[pl.ds(k, 16)][0]
        for t in range(DIN // 16):
            plsc.addupdate(acc.at[l, pl.ds(t * 16, 16)],
                           rows[k, pl.ds(t * 16, 16)])
        return c

    lax.fori_loop(0, GB, abody, 0)
    if shift:
        for g in range(128 // 16):
            pes[pl.ds(g * 16, 16)] = pes[pl.ds(GB + g * 16, 16)]
            plr[pl.ds(g * 16, 16)] = plr[pl.ds(GB + g * 16, 16)]
    return pcnt - GB


NRING = 6


def _owner_scan(table, eshbm, erhbm, cnthbm, cntv, pes, plr, rows, ebuf,
                rbuf, sem, acc, mylo, nrows, stride, nsup_all, lane, cid):
    pltpu.sync_copy(cnthbm.at[pl.ds(cid * NS * NCNT, NS * NCNT)],
                    cntv.at[pl.ds(0, NS * NCNT)])
    cbase = cid * NS * nsup_all * stride
    nreg = NS * nsup_all

    def issue(rid, slot):
        pltpu.async_copy(eshbm.at[pl.ds(cbase + rid * stride, stride)],
                         ebuf.at[pl.ds(slot * stride, stride)], sem)
        pltpu.async_copy(erhbm.at[pl.ds(cbase + rid * stride, stride)],
                         rbuf.at[pl.ds(slot * stride, stride)], sem)

    def wait(rid, slot):
        pltpu.make_async_copy(
            eshbm.at[pl.ds(cbase + rid * stride, stride)],
            ebuf.at[pl.ds(slot * stride, stride)], sem).wait()
        pltpu.make_async_copy(
            erhbm.at[pl.ds(cbase + rid * stride, stride)],
            rbuf.at[pl.ds(slot * stride, stride)], sem).wait()

    for r in range(NRING):
        issue(r, r)

    def region_body(rid, pcnt):
        slot = lax.rem(rid, NRING)
        wait(rid, slot)
        w = lax.div(rid, nsup_all)
        sch = rid - w * nsup_all
        cnt = cntv[pl.ds(w * NCNT + sch, 16)][0]
        groups = (cnt + 15) // 16

        def gbody(g, pcnt):
            lv = rbuf[pl.ds(slot * stride + g * 16, 16)] - mylo
            m = (lv >= 0) & (lv < nrows)
            plsc.store_compressed(plr.at[pl.ds(pcnt, 16)], lv, mask=m)
            plsc.store_compressed(pes.at[pl.ds(pcnt, 16)],
                                  ebuf[pl.ds(slot * stride + g * 16, 16)],
                                  mask=m)
            pcnt = pcnt + jnp.sum(m.astype(jnp.int32))
            return lax.cond(pcnt >= GB,
                            lambda: _fire(table, acc, pes, plr, rows, pcnt,
                                          True),
                            lambda: pcnt)

        pcnt = lax.fori_loop(0, groups, gbody, pcnt)
        nxt = rid + NRING

        @pl.when(nxt < nreg)
        def _reissue():
            issue(nxt, slot)

        return pcnt

    pcnt = lax.fori_loop(0, nreg, region_body, jnp.int32(0))
    for t in range(GB // 16):
        pes[pl.ds(pcnt + t * 16, 16)] = lane + t * 16
        plr[pl.ds(pcnt + t * 16, 16)] = nrows + lane + t * 16
    lax.cond(pcnt > 0,
             lambda: _fire(table, acc, pes, plr, rows, pcnt, False),
             lambda: pcnt)


def _sc_conv1_body(xp, srce, dste, out, eshbm, erhbm, cnthbm, sbuf, dbuf, es,
                   er, cbuf, cntv, pes, plr, rows, ebuf, rbuf, sem, spansem,
                   flushsem, acc):
    cid = lax.axis_index("c")
    sid = lax.axis_index("s")
    lane = lax.iota(jnp.int32, 16)
    for p in range(2):
        qid = cid * 2 + p
        qlo = qid * QN1

        cnts = []

        def keep(sv, dv, es_, er_, cnt):
            lv = dv - qlo
            m = (lv >= 0) & (lv < QN1)
            plsc.store_compressed(es_.at[pl.ds(cnt, 16)], sv, mask=m)
            plsc.store_compressed(er_.at[pl.ds(cnt, 16)], lv, mask=m)
            return cnt + jnp.sum(m.astype(jnp.int32))

        def span_issue(sch):
            nedge = SUP if sch < NSUP else TAIL
            sd = (sch % 2) * SUP
            base = sid * EPW + sch * SUP
            pltpu.async_copy(srce.at[pl.ds(base, nedge)],
                             sbuf.at[pl.ds(sd, nedge)], spansem)
            pltpu.async_copy(dste.at[pl.ds(base, nedge)],
                             dbuf.at[pl.ds(sd, nedge)], spansem)

        def span_wait(sch):
            nedge = SUP if sch < NSUP else TAIL
            sd = (sch % 2) * SUP
            base = sid * EPW + sch * SUP
            pltpu.make_async_copy(srce.at[pl.ds(base, nedge)],
                                  sbuf.at[pl.ds(sd, nedge)], spansem).wait()
            pltpu.make_async_copy(dste.at[pl.ds(base, nedge)],
                                  dbuf.at[pl.ds(sd, nedge)], spansem).wait()

        def flush_async(sch, cnt):
            fd = (sch % 2) * RS1
            for t in range(GB // 16):
                es[pl.ds(fd + cnt + t * 16, 16)] = lane + t * 16
                er[pl.ds(fd + cnt + t * 16, 16)] = jnp.full((16,), 1 << 20,
                                                            jnp.int32)
            rb = ((cid * NS + sid) * (NSUP + 1) + sch) * RS1
            pltpu.async_copy(es.at[pl.ds(fd, RS1)],
                             eshbm.at[pl.ds(rb, RS1)], flushsem)
            pltpu.async_copy(er.at[pl.ds(fd, RS1)],
                             erhbm.at[pl.ds(rb, RS1)], flushsem)

        def flush_wait(sch):
            fd = (sch % 2) * RS1
            rb = ((cid * NS + sid) * (NSUP + 1) + sch) * RS1
            pltpu.make_async_copy(es.at[pl.ds(fd, RS1)],
                                  eshbm.at[pl.ds(rb, RS1)], flushsem).wait()
            pltpu.make_async_copy(er.at[pl.ds(fd, RS1)],
                                  erhbm.at[pl.ds(rb, RS1)], flushsem).wait()

        span_issue(0)
        for sch in range(NSUP + 1):
            nedge = SUP if sch < NSUP else TAIL
            sd = (sch % 2) * SUP
            span_wait(sch)
            if sch + 1 <= NSUP:
                span_issue(sch + 1)
            if sch >= 2:
                flush_wait(sch - 2)
            cnt = _compact_chunk(sbuf.at[pl.ds(sd, SUP)],
                                 dbuf.at[pl.ds(sd, SUP)],
                                 es.at[pl.ds((sch % 2) * RS1, RS1)],
                                 er.at[pl.ds((sch % 2) * RS1, RS1)],
                                 nedge // 16, keep, jnp.int32(0))
            flush_async(sch, cnt)
            cnts.append(cnt)
        flush_wait(NSUP - 1)
        flush_wait(NSUP)
        _store_cnts(cbuf, cnts, lane)
        pltpu.sync_copy(cbuf,
                        cnthbm.at[pl.ds((cid * NS + sid) * NCNT, NCNT)])
        plsc.subcore_barrier()
        _zero_acc(acc, CROW1 + GB)
        _owner_scan(xp, eshbm, erhbm, cnthbm, cntv, pes, plr, rows, ebuf,
                    rbuf, sem, acc, sid * CROW1, CROW1, RS1, NSUP + 1, lane,
                    cid)
        pltpu.sync_copy(acc.at[pl.ds(0, CROW1)],
                        out.at[qid, pl.ds(sid * CROW1, CROW1)])
        plsc.subcore_barrier()


@functools.cache
def _get_sc_conv1():
    return pl.kernel(
        _sc_conv1_body,
        out_type=[
            jax.ShapeDtypeStruct((4, QN1, DIN), jnp.float32),
            jax.ShapeDtypeStruct((NC * NS * (NSUP + 1) * RS1,), jnp.int32),
            jax.ShapeDtypeStruct((NC * NS * (NSUP + 1) * RS1,), jnp.int32),
            jax.ShapeDtypeStruct((NC * NS * NCNT,), jnp.int32),
        ],
        mesh=plsc.VectorSubcoreMesh(core_axis_name="c", subcore_axis_name="s",
                                    num_cores=NC, num_subcores=NS),
        compiler_params=pltpu.CompilerParams(needs_layout_passes=False),
        scratch_types=[
            pltpu.VMEM((2 * SUP,), jnp.int32),
            pltpu.VMEM((2 * SUP,), jnp.int32),
            pltpu.VMEM((2 * RS1,), jnp.int32),
            pltpu.VMEM((2 * RS1,), jnp.int32),
            pltpu.VMEM((NCNT,), jnp.int32),
            pltpu.VMEM((NS * NCNT + 16,), jnp.int32),
            pltpu.VMEM((320,), jnp.int32),
            pltpu.VMEM((320,), jnp.int32),
            pltpu.VMEM((GB, DIN), jnp.float32),
            pltpu.VMEM((NRING * RS1,), jnp.int32),
            pltpu.VMEM((NRING * RS1,), jnp.int32),
            pltpu.SemaphoreType.DMA,
            pltpu.SemaphoreType.DMA,
            pltpu.SemaphoreType.DMA,
            pltpu.VMEM((CROW1 + GB, DIN), jnp.float32),
        ],
    )


def _sc_conv1(x_p, src, dst):
    return _get_sc_conv1()(x_p, src, dst)[0]


def _sc_conv2_body(hs2v, srce, dste, nidx, out, eshbm, erhbm, cnthbm, nv,
                   sbuf, dbuf, es, er, cbuf, cntv, pes, plr, rows, ebuf,
                   rbuf, sem, spansem, flushsem, acc):
    cid = lax.axis_index("c")
    sid = lax.axis_index("s")
    lane = lax.iota(jnp.int32, 16)
    pltpu.sync_copy(nidx, nv)
    for p in range(2):
        qid = cid * 2 + p
        qlo = qid * QN2
        cnts = []

        def keep(sv, dv, es_, er_, cnt):
            ns = plsc.load_gather(nv, [sv])
            nd = plsc.load_gather(nv, [dv]) - qlo
            m = (ns >= 0) & (nd >= 0) & (nd < QN2)
            c = jnp.sum(m.astype(jnp.int32))
            plsc.store_compressed(es_.at[pl.ds(cnt, 16)], 2 * sv, mask=m)
            plsc.store_compressed(er_.at[pl.ds(cnt, 16)], 2 * nd, mask=m)
            cnt = cnt + c
            plsc.store_compressed(es_.at[pl.ds(cnt, 16)], 2 * sv + 1, mask=m)
            plsc.store_compressed(er_.at[pl.ds(cnt, 16)], 2 * nd + 1, mask=m)
            return cnt + c

        def span_issue(sch):
            nedge = SUP if sch < NSUP else TAIL
            sd = (sch % 2) * SUP
            base = sid * EPW + sch * SUP
            pltpu.async_copy(srce.at[pl.ds(base, nedge)],
                             sbuf.at[pl.ds(sd, nedge)], spansem)
            pltpu.async_copy(dste.at[pl.ds(base, nedge)],
                             dbuf.at[pl.ds(sd, nedge)], spansem)

        def span_wait(sch):
            nedge = SUP if sch < NSUP else TAIL
            sd = (sch % 2) * SUP
            base = sid * EPW + sch * SUP
            pltpu.make_async_copy(srce.at[pl.ds(base, nedge)],
                                  sbuf.at[pl.ds(sd, nedge)], spansem).wait()
            pltpu.make_async_copy(dste.at[pl.ds(base, nedge)],
                                  dbuf.at[pl.ds(sd, nedge)], spansem).wait()

        def flush_async(sch, cnt):
            fd = (sch % 2) * RSS
            for t in range(GB // 16):
                es[pl.ds(fd + cnt + t * 16, 16)] = lane + t * 16
                er[pl.ds(fd + cnt + t * 16, 16)] = jnp.full((16,), 1 << 20,
                                                            jnp.int32)
            rb = ((cid * NS + sid) * (NSUP + 2) + sch) * RSS
            pltpu.async_copy(es.at[pl.ds(fd, RS2)],
                             eshbm.at[pl.ds(rb, RS2)], flushsem)
            pltpu.async_copy(er.at[pl.ds(fd, RS2)],
                             erhbm.at[pl.ds(rb, RS2)], flushsem)

        def flush_wait(sch):
            fd = (sch % 2) * RSS
            rb = ((cid * NS + sid) * (NSUP + 2) + sch) * RSS
            pltpu.make_async_copy(es.at[pl.ds(fd, RS2)],
                                  eshbm.at[pl.ds(rb, RS2)], flushsem).wait()
            pltpu.make_async_copy(er.at[pl.ds(fd, RS2)],
                                  erhbm.at[pl.ds(rb, RS2)], flushsem).wait()

        span_issue(0)
        for sch in range(NSUP + 1):
            nedge = SUP if sch < NSUP else TAIL
            sd = (sch % 2) * SUP
            span_wait(sch)
            if sch + 1 <= NSUP:
                span_issue(sch + 1)
            if sch >= 2:
                flush_wait(sch - 2)
            cnt = _compact_chunk(sbuf.at[pl.ds(sd, SUP)],
                                 dbuf.at[pl.ds(sd, SUP)],
                                 es.at[pl.ds((sch % 2) * RSS, RSS)],
                                 er.at[pl.ds((sch % 2) * RSS, RSS)],
                                 nedge // 16, keep, jnp.int32(0))
            flush_async(sch, cnt)
            cnts.append(cnt)
        flush_wait(NSUP - 1)
        flush_wait(NSUP)

        def sbody(i, cnt):
            base = sid * NPW + i * 16
            nd = nv[pl.ds(base, 16)] - qlo
            m = (nd >= 0) & (nd < QN2)
            c = jnp.sum(m.astype(jnp.int32))
            plsc.store_compressed(es.at[pl.ds(cnt, 16)],
                                  2 * (lane + base), mask=m)
            plsc.store_compressed(er.at[pl.ds(cnt, 16)], 2 * nd, mask=m)
            cnt = cnt + c
            plsc.store_compressed(es.at[pl.ds(cnt, 16)],
                                  2 * (lane + base) + 1, mask=m)
            plsc.store_compressed(er.at[pl.ds(cnt, 16)], 2 * nd + 1, mask=m)
            return cnt + c

        cnt = lax.fori_loop(0, NPW // 16, sbody, jnp.int32(0))
        _flush(es.at[pl.ds(0, RSS)], er.at[pl.ds(0, RSS)], eshbm, erhbm,
               ((cid * NS + sid) * (NSUP + 2) + NSUP + 1) * RSS, cnt,
               lane, RSS)
        cnts.append(cnt)
        _store_cnts(cbuf, cnts, lane)
        pltpu.sync_copy(cbuf,
                        cnthbm.at[pl.ds((cid * NS + sid) * NCNT, NCNT)])
        plsc.subcore_barrier()
        _zero_acc(acc, CROW2 + GB)
        _owner_scan(hs2v, eshbm, erhbm, cnthbm, cntv, pes, plr, rows, ebuf,
                    rbuf, sem, acc, sid * CROW2, CROW2, RSS, NSUP + 2, lane,
                    cid)
        pltpu.sync_copy(acc.at[pl.ds(0, CROW2)],
                        out.at[qid, pl.ds(sid * CROW2, CROW2)])
        plsc.subcore_barrier()


@functools.cache
def _get_sc_conv2():
    return pl.kernel(
        _sc_conv2_body,
        out_type=[
            jax.ShapeDtypeStruct((4, 2 * QN2, DIN), jnp.float32),
            jax.ShapeDtypeStruct((NC * NS * (NSUP + 2) * RSS,), jnp.int32),
            jax.ShapeDtypeStruct((NC * NS * (NSUP + 2) * RSS,), jnp.int32),
            jax.ShapeDtypeStruct((NC * NS * NCNT,), jnp.int32),
        ],
        mesh=plsc.VectorSubcoreMesh(core_axis_name="c", subcore_axis_name="s",
                                    num_cores=NC, num_subcores=NS),
        compiler_params=pltpu.CompilerParams(needs_layout_passes=False),
        scratch_types=[
            pltpu.VMEM((NP,), jnp.int32),
            pltpu.VMEM((2 * SUP,), jnp.int32),
            pltpu.VMEM((2 * SUP,), jnp.int32),
            pltpu.VMEM((2 * RSS,), jnp.int32),
            pltpu.VMEM((2 * RSS,), jnp.int32),
            pltpu.VMEM((NCNT,), jnp.int32),
            pltpu.VMEM((NS * NCNT + 16,), jnp.int32),
            pltpu.VMEM((320,), jnp.int32),
            pltpu.VMEM((320,), jnp.int32),
            pltpu.VMEM((GB, DIN), jnp.float32),
            pltpu.VMEM((NRING * RSS,), jnp.int32),
            pltpu.VMEM((NRING * RSS,), jnp.int32),
            pltpu.SemaphoreType.DMA,
            pltpu.SemaphoreType.DMA,
            pltpu.SemaphoreType.DMA,
            pltpu.VMEM((CROW2 + GB, DIN), jnp.float32),
        ],
    )


def _sc_conv2(hs1, src, dst, nidx):
    out = _get_sc_conv2()(hs1.reshape(2 * NP, DIN), src, dst, nidx)[0]
    return out.reshape(4, QN2, H)


def _keys_from_score(sc, rid, nreal):
    bits = lax.bitcast_convert_type(sc, jnp.int32)
    key = jnp.where(bits < 0, bits ^ jnp.int32(0x7FFFFFFF), bits)
    key = key + jnp.int32(KEY_BIAS)
    return jnp.where(rid < nreal, key, 0)


def _bisect_tau(kk, k):
    def bit_body(t, tcur):
        cand = tcur | (jnp.int32(1) << (jnp.int32(30) - t))
        cnt = jnp.sum((kk >= cand).astype(jnp.int32))
        return jnp.where(cnt >= k, cand, tcur)

    return lax.fori_loop(0, 31, bit_body, jnp.int32(0))


def _bisect_cut(eq, rid, need):
    def bit_body(t, cur):
        cand = cur | (jnp.int32(1) << (jnp.int32(30) - t))
        cnt = jnp.sum((eq & (rid < cand)).astype(jnp.int32))
        return jnp.where(cnt <= need, cand, cur)

    return lax.fori_loop(0, 31, bit_body, jnp.int32(0))


def _sel_consts(kk, rid3, k):
    tau = _bisect_tau(kk, k)
    cgt = jnp.sum((kk > tau).astype(jnp.int32))
    need = jnp.int32(k) - cgt
    cut = _bisect_cut(kk == tau, rid3, need)
    return tau, cut


def _mlp1_body(xb, ab, W1r, b1r, W2r, b2r, hout, statout, ssum, ssq):
    i = pl.program_id(0)
    h0 = xb[...] + ab[0]
    z1 = jnp.dot(h0, W1r[...], preferred_element_type=jnp.float32) + b1r[...]
    g1 = jax.nn.gelu(z1)
    z2 = jnp.dot(g1, W2r[...], preferred_element_type=jnp.float32) + b2r[...]
    hg = jax.nn.gelu(z2)
    rid = i * RB + lax.broadcasted_iota(jnp.int32, (RB, 1), 0)
    hm = jnp.where(rid < N, hg, 0.0)
    hout[...] = hm

    @pl.when(i == 0)
    def _init():
        ssum[...] = jnp.zeros((1, H), jnp.float32)
        ssq[...] = jnp.zeros((1, H), jnp.float32)

    ssum[...] += jnp.sum(hm, axis=0, keepdims=True)
    ssq[...] += jnp.sum(hm * hm, axis=0, keepdims=True)

    @pl.when(i == NB1 - 1)
    def _fin():
        statout[0:1, :] = ssum[...]
        statout[1:2, :] = ssq[...]


def _tc_mlp1(x_p, agg, W1, b1, W2, b2):
    return pl.pallas_call(
        _mlp1_body,
        grid=(NB1,),
        in_specs=[
            pl.BlockSpec((RB, DIN), lambda i: (i, 0)),
            pl.BlockSpec((1, RB, DIN), lambda i: (i, 0, 0)),
            pl.BlockSpec((DIN, H), lambda i: (0, 0)),
            pl.BlockSpec((1, H), lambda i: (0, 0)),
            pl.BlockSpec((H, H), lambda i: (0, 0)),
            pl.BlockSpec((1, H), lambda i: (0, 0)),
        ],
        out_specs=[
            pl.BlockSpec((RB, H), lambda i: (i, 0)),
            pl.BlockSpec((2, H), lambda i: (0, 0)),
        ],
        out_shape=[
            jax.ShapeDtypeStruct((NP, H), jnp.float32),
            jax.ShapeDtypeStruct((2, H), jnp.float32),
        ],
        scratch_shapes=[
            pltpu.VMEM((1, H), jnp.float32),
            pltpu.VMEM((1, H), jnp.float32),
        ],
    )(x_p, agg.reshape(NB1, RB, DIN), W1, b1[None, :], W2,
      b2[None, :])


def _score1_body(hb, q1r, c1r, a1r, d1r, hsout, nout, scout, tout, keys):
    i = pl.program_id(0)

    @pl.when(i < NB1)
    def _compute():
        h = hb[...]
        z = jnp.dot(h, q1r[...], preferred_element_type=jnp.float32)
        sc = jnp.tanh(z + c1r[0, 0])
        hsout[...] = (h * a1r[...] + d1r[...]) * sc
        scout[...] = sc
        rid = i * RB + lax.broadcasted_iota(jnp.int32, (RB, 1), 0)
        keys[i] = _keys_from_score(sc, rid, N)

    @pl.when(i == NB1)
    def _select():
        kk = keys[...]
        rid3 = (lax.broadcasted_iota(jnp.int32, (NB1, RB, 1), 0) * RB
                + lax.broadcasted_iota(jnp.int32, (NB1, RB, 1), 1))
        tau, cut = _sel_consts(kk, rid3, K1)
        sel = (kk > tau) | ((kk == tau) & (rid3 < cut))
        i0 = lax.broadcasted_iota(jnp.int32, (RB, RB), 0)
        i1 = lax.broadcasted_iota(jnp.int32, (RB, RB), 1)
        ltri = jnp.where(i0 > i1, 1.0, 0.0)
        off = jnp.float32(0.0)
        for g in range(NB1):
            selg = sel[g].astype(jnp.float32)
            pre = jnp.dot(ltri, selg, preferred_element_type=jnp.float32)
            nout[g] = jnp.where(sel[g], (pre + off).astype(jnp.int32), -1)
            off = off + jnp.sum(selg)
        tout[0:1, 0:1] = jnp.full((1, 1), tau, jnp.int32)
        tout[0:1, 1:2] = jnp.full((1, 1), cut, jnp.int32)


def _tc_score1(h1g, q1, c1, a1v, d1v):
    return pl.pallas_call(
        _score1_body,
        grid=(NB1 + 1,),
        in_specs=[
            pl.BlockSpec((RB, H), lambda i: (jnp.minimum(i, NB1 - 1), 0)),
            pl.BlockSpec((H, 1), lambda i: (0, 0)),
            pl.BlockSpec((1, 1), lambda i: (0, 0)),
            pl.BlockSpec((1, H), lambda i: (0, 0)),
            pl.BlockSpec((1, H), lambda i: (0, 0)),
        ],
        out_specs=[
            pl.BlockSpec((RB, H), lambda i: (jnp.minimum(i, NB1 - 1), 0)),
            pl.BlockSpec((NB1, RB, 1), lambda i: (0, 0, 0)),
            pl.BlockSpec((RB, 1), lambda i: (jnp.minimum(i, NB1 - 1), 0)),
            pl.BlockSpec((1, 128), lambda i: (0, 0)),
        ],
        out_shape=[
            jax.ShapeDtypeStruct((NP, H), jnp.float32),
            jax.ShapeDtypeStruct((NB1, RB, 1), jnp.int32),
            jax.ShapeDtypeStruct((NP, 1), jnp.float32),
            jax.ShapeDtypeStruct((1, 128), jnp.int32),
        ],
        scratch_shapes=[
            pltpu.VMEM((NB1, RB, 1), jnp.int32),
        ],
    )(h1g, q1[:, None], c1.reshape(1, 1), a1v[None, :], d1v[None, :])


def _mlp2_body(hsb, scb, tselr, dqb, W3r, b3r, W4r, b4r, hout, statout,
               x1out, smax, ssum, s2sum, s2sq):
    i = pl.program_id(0)

    @pl.when(i == 0)
    def _init1():
        smax[...] = jnp.full((1, H), -jnp.inf, jnp.float32)
        ssum[...] = jnp.zeros((1, H), jnp.float32)

    @pl.when(i < NB1)
    def _pool1():
        rid = i * RB + lax.broadcasted_iota(jnp.int32, (RB, 1), 0)
        key = _keys_from_score(scb[...], rid, N)
        tau = tselr[0, 0]
        cut = tselr[0, 1]
        sel = (key > tau) | ((key == tau) & (rid < cut))
        hs = hsb[...]
        smax[...] = jnp.maximum(
            smax[...], jnp.max(jnp.where(sel, hs, -jnp.inf), axis=0,
                               keepdims=True))
        ssum[...] += jnp.sum(jnp.where(sel, hs, 0.0), axis=0, keepdims=True)

    @pl.when(i == NB1)
    def _init2():
        s2sum[...] = jnp.zeros((1, H), jnp.float32)
        s2sq[...] = jnp.zeros((1, H), jnp.float32)

    @pl.when(i >= NB1)
    def _mlp():
        j = i - NB1
        h0 = dqb[0]
        z3 = jnp.dot(h0, W3r[...], preferred_element_type=jnp.float32) + b3r[...]
        g3 = jax.nn.gelu(z3)
        z4 = jnp.dot(g3, W4r[...], preferred_element_type=jnp.float32) + b4r[...]
        hg = jax.nn.gelu(z4)
        rid = j * RB2 + lax.broadcasted_iota(jnp.int32, (RB2, 1), 0)
        hm = jnp.where(rid < K1, hg, 0.0)
        hout[...] = hm
        s2sum[...] += jnp.sum(hm, axis=0, keepdims=True)
        s2sq[...] += jnp.sum(hm * hm, axis=0, keepdims=True)

    @pl.when(i == NB1 + NB2 - 1)
    def _fin():
        statout[0:1, :] = s2sum[...]
        statout[1:2, :] = s2sq[...]
        x1out[0:1, :] = smax[...]
        x1out[1:2, :] = ssum[...]


def _tc_mlp2(hs1, scol, tsel, aggq, W3, b3, W4, b4):
    nb = NB1 + NB2
    return pl.pallas_call(
        _mlp2_body,
        grid=(nb,),
        in_specs=[
            pl.BlockSpec((RB, H), lambda i: (jnp.minimum(i, NB1 - 1), 0)),
            pl.BlockSpec((RB, 1), lambda i: (jnp.minimum(i, NB1 - 1), 0)),
            pl.BlockSpec((1, 128), lambda i: (0, 0)),
            pl.BlockSpec((1, RB2, H),
                         lambda i: (jnp.clip(i - NB1, 0, NB2 - 1), 0, 0)),
            pl.BlockSpec((H, H), lambda i: (0, 0)),
            pl.BlockSpec((1, H), lambda i: (0, 0)),
            pl.BlockSpec((H, H), lambda i: (0, 0)),
            pl.BlockSpec((1, H), lambda i: (0, 0)),
        ],
        out_specs=[
            pl.BlockSpec((RB2, H),
                         lambda i: (jnp.clip(i - NB1, 0, NB2 - 1), 0)),
            pl.BlockSpec((2, H), lambda i: (0, 0)),
            pl.BlockSpec((2, H), lambda i: (0, 0)),
        ],
        out_shape=[
            jax.ShapeDtypeStruct((K1P, H), jnp.float32),
            jax.ShapeDtypeStruct((2, H), jnp.float32),
            jax.ShapeDtypeStruct((2, H), jnp.float32),
        ],
        scratch_shapes=[
            pltpu.VMEM((1, H), jnp.float32),
            pltpu.VMEM((1, H), jnp.float32),
            pltpu.VMEM((1, H), jnp.float32),
            pltpu.VMEM((1, H), jnp.float32),
        ],
    )(hs1, scol, tsel, aggq.reshape(4 * (QN2 // RB2), RB2, H)[:NB2],
      W3, b3[None, :], W4, b4[None, :])


NBF = K1P // RB


def _final_body(hb, q2r, c2r, a2r, d2r, x1r, Wlr, blr, oout, keys, hs2s):
    i = pl.program_id(0)

    @pl.when(i < NBF)
    def _compute():
        h = hb[...]
        z = jnp.dot(h, q2r[...], preferred_element_type=jnp.float32)
        sc = jnp.tanh(z + c2r[0, 0])
        rid = i * RB + lax.broadcasted_iota(jnp.int32, (RB, 1), 0)
        keys[i] = _keys_from_score(sc, rid, K1)
        hs2s[i] = (h * a2r[...] + d2r[...]) * sc

    @pl.when(i == NBF)
    def _select():
        kk = keys[...]
        rid3 = (lax.broadcasted_iota(jnp.int32, (NBF, RB, 1), 0) * RB
                + lax.broadcasted_iota(jnp.int32, (NBF, RB, 1), 1))
        tau, cut = _sel_consts(kk, rid3, K2)
        mx2 = jnp.full((1, H), -jnp.inf, jnp.float32)
        sm2 = jnp.zeros((1, H), jnp.float32)
        for g in range(NBF):
            ridg = g * RB + lax.broadcasted_iota(jnp.int32, (RB, 1), 0)
            kg = keys[g]
            selg = (kg > tau) | ((kg == tau) & (ridg < cut))
            hsg = hs2s[g]
            mx2 = jnp.maximum(
                mx2, jnp.max(jnp.where(selg, hsg, -jnp.inf), axis=0,
                             keepdims=True))
            sm2 = sm2 + jnp.sum(jnp.where(selg, hsg, 0.0), axis=0,
                                keepdims=True)
        x1 = jnp.concatenate([x1r[0:1, :], x1r[1:2, :] / K1], axis=1)
        x2 = jnp.concatenate([mx2, sm2 / K2], axis=1)
        xt = x1 + x2
        oout[...] = jnp.dot(xt, Wlr[...],
                            preferred_element_type=jnp.float32) + blr[...]


def _tc_final(h2g, q2, c2, a2v, d2v, x1p, Wl, bl):
    return pl.pallas_call(
        _final_body,
        grid=(NBF + 1,),
        in_specs=[
            pl.BlockSpec((RB, H), lambda i: (jnp.minimum(i, NBF - 1), 0)),
            pl.BlockSpec((H, 1), lambda i: (0, 0)),
            pl.BlockSpec((1, 1), lambda i: (0, 0)),
            pl.BlockSpec((1, H), lambda i: (0, 0)),
            pl.BlockSpec((1, H), lambda i: (0, 0)),
            pl.BlockSpec((2, H), lambda i: (0, 0)),
            pl.BlockSpec((2 * H, H), lambda i: (0, 0)),
            pl.BlockSpec((1, H), lambda i: (0, 0)),
        ],
        out_specs=pl.BlockSpec((1, H), lambda i: (0, 0)),
        out_shape=jax.ShapeDtypeStruct((1, H), jnp.float32),
        scratch_shapes=[
            pltpu.VMEM((NBF, RB, 1), jnp.int32),
            pltpu.VMEM((NBF, RB, H), jnp.float32),
        ],
    )(h2g, q2[:, None], c2.reshape(1, 1), a2v[None, :], d2v[None, :], x1p, Wl,
      bl[None, :])


def kernel(x, edge_index, batch, W1, b1, W2, b2, gamma1, beta1, p1, W3, b3,
           W4, b4, gamma2, beta2, p2, Wl, bl):
    src = edge_index[0]
    dst = edge_index[1]
    x_p = jnp.pad(x, ((0, NP - N), (0, 0)))

    agg = _sc_conv1(x_p, src, dst)
    h1g, st1 = _tc_mlp1(x_p, agg, W1, b1, W2, b2)

    mean1 = st1[0] / N
    var1 = st1[1] / N - mean1 * mean1
    a1v = gamma1 * lax.rsqrt(var1 + 1e-5)
    d1v = beta1 - mean1 * a1v
    nrm1 = jnp.sqrt(jnp.sum(p1 * p1))
    q1 = a1v * p1 / nrm1
    c1 = jnp.sum(d1v * p1) / nrm1

    hs1, nidx, scol, tsel = _tc_score1(h1g, q1, c1, a1v, d1v)

    agg2 = _sc_conv2(hs1, src, dst, nidx.reshape(NP))
    h2g, st2, x1p = _tc_mlp2(hs1, scol, tsel, agg2, W3, b3, W4, b4)

    mean2 = st2[0] / K1
    var2 = st2[1] / K1 - mean2 * mean2
    a2v = gamma2 * lax.rsqrt(var2 + 1e-5)
    d2v = beta2 - mean2 * a2v
    nrm2 = jnp.sqrt(jnp.sum(p2 * p2))
    q2 = a2v * p2 / nrm2
    c2 = jnp.sum(d2v * p2) / nrm2

    return _tc_final(h2g, q2, c2, a2v, d2v, x1p, Wl, bl)
